# Initial kernel scaffold; baseline (speedup 1.0000x reference)
import functools

import jax
import jax.numpy as jnp
from jax import lax
from jax.experimental import pallas as pl
from jax.experimental.pallas import tpu as pltpu
from jax.experimental.pallas import tpu_sc as plsc

N = 10000
E = 320000
C = 64
L = 4
NC = 2
NS = 16
NW = NC * NS
EPW = E // NW
CHUNK = 80
NCHUNK = EPW // CHUNK
ROWS_PT = N // NS
ZROWS = 125


def _pre_body(x_ref, w_ref, asv_ref, adv_ref, we_ref, ae_ref, ea_ref,
              xl_ref, asrc_ref, adst_ref, eac_ref):
    xl = jnp.dot(x_ref[...], w_ref[...], preferred_element_type=jnp.float32)
    xl_ref[...] = xl
    asrc_ref[...] = jnp.dot(xl, asv_ref[...], preferred_element_type=jnp.float32)
    adst_ref[...] = jnp.dot(xl, adv_ref[...], preferred_element_type=jnp.float32)
    c = jnp.sum(we_ref[...] * ae_ref[...])
    eac_ref[...] = ea_ref[...] * c


_pre_call = pl.pallas_call(
    _pre_body,
    out_shape=[
        jax.ShapeDtypeStruct((N, C), jnp.float32),
        jax.ShapeDtypeStruct((N, 1), jnp.float32),
        jax.ShapeDtypeStruct((N, 1), jnp.float32),
        jax.ShapeDtypeStruct((E // 128, 128), jnp.float32),
    ],
)


def _post_body(oa_ref, ob_ref, dp_ref, gp_ref, sp_ref, xl_ref, as_ref, ad_ref,
               b_ref, wo_ref, bo_ref, ones_ref, res_ref):
    cdims = (((0,), (0,)), ((), ()))
    ones = ones_ref[...]
    dn = lax.dot_general(dp_ref[...], ones, cdims, preferred_element_type=jnp.float32)
    dg = lax.dot_general(gp_ref[...], ones, cdims, preferred_element_type=jnp.float32)
    sa = lax.dot_general(sp_ref[...], ones, cdims, preferred_element_type=jnp.float32)
    loop_c = sa / jnp.maximum(dg, 1.0)
    a_self = as_ref[...] + ad_ref[...] + loop_c
    a_self = jnp.where(a_self >= 0.0, a_self, 0.2 * a_self)
    ex_self = jnp.exp(a_self)
    xl = xl_ref[...]
    outu = oa_ref[...] + ob_ref[...] + ex_self * xl
    out = outu / jnp.maximum(dn + ex_self, 1e-16) + b_ref[...]
    res = lax.dot_general(wo_ref[...], out, (((1,), (1,)), ((), ())),
                          preferred_element_type=jnp.float32)
    res_ref[...] = res + bo_ref[...]


_post_call = pl.pallas_call(
    _post_body,
    out_shape=jax.ShapeDtypeStruct((L, N), jnp.float32),
)


_sc_mesh = plsc.VectorSubcoreMesh(core_axis_name="c", subcore_axis_name="s")


@functools.partial(
    pl.kernel,
    out_type=[
        jax.ShapeDtypeStruct((NC, N, C), jnp.float32),
        jax.ShapeDtypeStruct((NW, N), jnp.float32),
        jax.ShapeDtypeStruct((NW, N), jnp.float32),
        jax.ShapeDtypeStruct((NW, N), jnp.float32),
    ],
    mesh=_sc_mesh,
    scratch_types=[
        pltpu.VMEM((N,), jnp.float32),
        pltpu.VMEM((N,), jnp.float32),
        pltpu.VMEM((N,), jnp.float32),
        pltpu.VMEM((N,), jnp.float32),
        pltpu.VMEM((N,), jnp.float32),
        pltpu.VMEM((CHUNK,), jnp.int32),
        pltpu.VMEM((CHUNK,), jnp.int32),
        pltpu.VMEM((CHUNK,), jnp.float32),
        pltpu.VMEM((CHUNK, C), jnp.float32),
        pltpu.VMEM((CHUNK,), jnp.float32),
        pltpu.VMEM((ZROWS, C), jnp.float32),
        pltpu.VMEM_SHARED((N, C), jnp.float32),
        pltpu.SemaphoreType.DMA,
    ],
)
def _sc_edge(src_hbm, dst_hbm, eac_hbm, asrc_hbm, adst_hbm, xl_hbm,
             outu_hbm, denom_hbm, deg_hbm, sattr_hbm,
             asrc_v, adst_v, denom_v, deg_v, sattr_v,
             srcb, dstb, eacb, rowsb, exb, zbuf, outu_sh, sem):
    cid = lax.axis_index("c")
    sid = lax.axis_index("s")
    wid = sid * NC + cid
    ebase = wid * EPW

    pltpu.sync_copy(asrc_hbm, asrc_v)
    pltpu.sync_copy(adst_hbm, adst_v)

    zeros16 = jnp.zeros((16,), jnp.float32)
    ones16 = jnp.full((16,), 1.0, jnp.float32)

    def zero_locals(i, carry):
        denom_v[pl.ds(i * 16, 16)] = zeros16
        deg_v[pl.ds(i * 16, 16)] = zeros16
        sattr_v[pl.ds(i * 16, 16)] = zeros16
        return carry

    lax.fori_loop(0, N // 16, zero_locals, 0)

    def zero_zbuf(r, carry):
        for q in range(C // 16):
            zbuf[r, pl.ds(q * 16, 16)] = zeros16
        return carry

    lax.fori_loop(0, ZROWS, zero_zbuf, 0)

    def zero_shared(j, carry):
        pltpu.sync_copy(zbuf, outu_sh.at[pl.ds(sid * ROWS_PT + j * ZROWS, ZROWS), :])
        return carry

    lax.fori_loop(0, ROWS_PT // ZROWS, zero_shared, 0)
    plsc.subcore_barrier()

    def chunk_body(k, carry):
        base = ebase + k * CHUNK
        pltpu.sync_copy(src_hbm.at[pl.ds(base, CHUNK)], srcb)
        pltpu.sync_copy(dst_hbm.at[pl.ds(base, CHUNK)], dstb)
        pltpu.sync_copy(eac_hbm.at[pl.ds(base, CHUNK)], eacb)
        pltpu.async_copy(xl_hbm.at[srcb], rowsb, sem).wait()
        for g in range(CHUNK // 16):
            s16 = srcb[pl.ds(g * 16, 16)]
            d16 = dstb[pl.ds(g * 16, 16)]
            e16 = eacb[pl.ds(g * 16, 16)]
            av = plsc.load_gather(asrc_v, [s16])
            bv = plsc.load_gather(adst_v, [d16])
            al = av + bv + e16
            al = jnp.where(al >= 0.0, al, 0.2 * al)
            ex = jnp.exp(al)
            exb[pl.ds(g * 16, 16)] = ex
            plsc.addupdate_scatter(denom_v, [d16], ex)
            plsc.addupdate_scatter(deg_v, [d16], ones16)
            plsc.addupdate_scatter(sattr_v, [d16], e16)

        def scale_row(r, carry2):
            exs = exb[r]
            for q in range(C // 16):
                rowsb[r, pl.ds(q * 16, 16)] = rowsb[r, pl.ds(q * 16, 16)] * exs
            return carry2

        lax.fori_loop(0, CHUNK, scale_row, 0)
        pltpu.sync_copy(rowsb, outu_sh.at[dstb], add=True)
        return carry

    lax.fori_loop(0, NCHUNK, chunk_body, 0)

    plsc.subcore_barrier()
    pltpu.sync_copy(denom_v, denom_hbm.at[wid])
    pltpu.sync_copy(deg_v, deg_hbm.at[wid])
    pltpu.sync_copy(sattr_v, sattr_hbm.at[wid])
    pltpu.sync_copy(outu_sh.at[pl.ds(sid * ROWS_PT, ROWS_PT), :],
                    outu_hbm.at[cid, pl.ds(sid * ROWS_PT, ROWS_PT), :])


def kernel(x, edge_index, edge_attr, W, b, att_src, att_dst, W_edge, att_edge,
           W_out, b_out):
    src = edge_index[0].astype(jnp.int32)
    dst = edge_index[1].astype(jnp.int32)
    ea2 = edge_attr.reshape(E // 128, 128)
    xl, asrc2, adst2, eac2 = _pre_call(
        x, W, att_src[:, None], att_dst[:, None], W_edge, att_edge[None, :], ea2)
    eac = eac2.reshape(E)
    asrc = asrc2.reshape(N)
    adst = adst2.reshape(N)
    outu_p, denom_p, deg_p, sattr_p = _sc_edge(src, dst, eac, asrc, adst, xl)
    res = _post_call(outu_p[0], outu_p[1], denom_p, deg_p, sattr_p, xl,
                     asrc2, adst2, b[None, :], W_out, b_out[:, None],
                     jnp.ones((NW, 1), jnp.float32))
    return res[:, :, None]


# speedup vs baseline: 25.1848x; 25.1848x over previous
import functools

import jax
import jax.numpy as jnp
from jax import lax
from jax.experimental import pallas as pl
from jax.experimental.pallas import tpu as pltpu
from jax.experimental.pallas import tpu_sc as plsc

N = 10000
E = 320000
C = 64
L = 4
NC = 2
NS = 16
NW = NC * NS
EPW = E // NW
CHUNK = 80
NCHUNK = EPW // CHUNK
NP = 10240
ROWS_PT = NP // NS
ZROWS = 128


def _pre_body(x_ref, w_ref, asv_ref, adv_ref, we_ref, ae_ref, ea_ref,
              xl_ref, asrc_ref, adst_ref, eac_ref):
    xl = jnp.dot(x_ref[...], w_ref[...], preferred_element_type=jnp.float32)
    xl_ref[...] = xl
    asrc_ref[...] = jnp.dot(xl, asv_ref[...], preferred_element_type=jnp.float32)
    adst_ref[...] = jnp.dot(xl, adv_ref[...], preferred_element_type=jnp.float32)
    c = jnp.sum(we_ref[...] * ae_ref[...])
    eac_ref[...] = ea_ref[...] * c


_pre_call = pl.pallas_call(
    _pre_body,
    out_shape=[
        jax.ShapeDtypeStruct((N, C), jnp.float32),
        jax.ShapeDtypeStruct((N, 1), jnp.float32),
        jax.ShapeDtypeStruct((N, 1), jnp.float32),
        jax.ShapeDtypeStruct((E // 128, 128), jnp.float32),
    ],
)


def _post_body(oa_ref, ob_ref, dp_ref, gp_ref, sp_ref, xl_ref, as_ref, ad_ref,
               b_ref, wo_ref, bo_ref, ones_ref, res_ref):
    cdims = (((0,), (0,)), ((), ()))
    ones = ones_ref[...]
    dn = lax.dot_general(dp_ref[...], ones, cdims, preferred_element_type=jnp.float32)
    dg = lax.dot_general(gp_ref[...], ones, cdims, preferred_element_type=jnp.float32)
    sa = lax.dot_general(sp_ref[...], ones, cdims, preferred_element_type=jnp.float32)
    loop_c = sa / jnp.maximum(dg, 1.0)
    a_self = as_ref[...] + ad_ref[...] + loop_c
    a_self = jnp.where(a_self >= 0.0, a_self, 0.2 * a_self)
    ex_self = jnp.exp(a_self)
    xl = xl_ref[...]
    outu = oa_ref[...] + ob_ref[...] + ex_self * xl
    out = outu / jnp.maximum(dn + ex_self, 1e-16) + b_ref[...]
    res = lax.dot_general(wo_ref[...], out, (((1,), (1,)), ((), ())),
                          preferred_element_type=jnp.float32)
    res_ref[...] = res + bo_ref[...]


_post_call = pl.pallas_call(
    _post_body,
    out_shape=jax.ShapeDtypeStruct((L, N), jnp.float32),
)


_sc_mesh = plsc.VectorSubcoreMesh(core_axis_name="c", subcore_axis_name="s")


@functools.partial(
    pl.kernel,
    out_type=[
        jax.ShapeDtypeStruct((NC, NP, C), jnp.float32),
        jax.ShapeDtypeStruct((NW * NP,), jnp.float32),
        jax.ShapeDtypeStruct((NW * NP,), jnp.float32),
        jax.ShapeDtypeStruct((NW * NP,), jnp.float32),
    ],
    mesh=_sc_mesh,
    compiler_params=pltpu.CompilerParams(
        needs_layout_passes=False, use_tc_tiling_on_sc=False),
    scratch_types=[
        pltpu.VMEM((N,), jnp.float32),
        pltpu.VMEM((N,), jnp.float32),
        pltpu.VMEM((NP,), jnp.float32),
        pltpu.VMEM((NP,), jnp.float32),
        pltpu.VMEM((NP,), jnp.float32),
        pltpu.VMEM((CHUNK,), jnp.int32),
        pltpu.VMEM((CHUNK,), jnp.int32),
        pltpu.VMEM((CHUNK,), jnp.float32),
        pltpu.VMEM((CHUNK, C), jnp.float32),
        pltpu.VMEM((CHUNK,), jnp.float32),
        pltpu.VMEM((ZROWS, C), jnp.float32),
        pltpu.VMEM_SHARED((NP, C), jnp.float32),
        pltpu.SemaphoreType.DMA,
    ],
)
def _sc_edge(src_hbm, dst_hbm, eac_hbm, asrc_hbm, adst_hbm, xl_hbm,
             outu_hbm, denom_hbm, deg_hbm, sattr_hbm,
             asrc_v, adst_v, denom_v, deg_v, sattr_v,
             srcb, dstb, eacb, rowsb, exb, zbuf, outu_sh, sem):
    cid = lax.axis_index("c")
    sid = lax.axis_index("s")
    wid = sid * NC + cid
    ebase = wid * EPW

    pltpu.sync_copy(asrc_hbm, asrc_v)
    pltpu.sync_copy(adst_hbm, adst_v)

    zeros16 = jnp.zeros((16,), jnp.float32)
    ones16 = jnp.full((16,), 1.0, jnp.float32)

    def zero_locals(i, carry):
        denom_v[pl.ds(i * 16, 16)] = zeros16
        deg_v[pl.ds(i * 16, 16)] = zeros16
        sattr_v[pl.ds(i * 16, 16)] = zeros16
        return carry

    lax.fori_loop(0, NP // 16, zero_locals, 0)

    def zero_zbuf(r, carry):
        for q in range(C // 16):
            zbuf[r, pl.ds(q * 16, 16)] = zeros16
        return carry

    lax.fori_loop(0, ZROWS, zero_zbuf, 0)

    def zero_shared(j, carry):
        pltpu.sync_copy(zbuf, outu_sh.at[pl.ds(sid * ROWS_PT + j * ZROWS, ZROWS), :])
        return carry

    lax.fori_loop(0, ROWS_PT // ZROWS, zero_shared, 0)
    plsc.subcore_barrier()

    def chunk_body(k, carry):
        base = ebase + k * CHUNK
        pltpu.sync_copy(src_hbm.at[pl.ds(base, CHUNK)], srcb)
        pltpu.sync_copy(dst_hbm.at[pl.ds(base, CHUNK)], dstb)
        pltpu.sync_copy(eac_hbm.at[pl.ds(base, CHUNK)], eacb)
        pltpu.async_copy(xl_hbm.at[srcb], rowsb, sem).wait()
        for g in range(CHUNK // 16):
            s16 = srcb[pl.ds(g * 16, 16)]
            d16 = dstb[pl.ds(g * 16, 16)]
            e16 = eacb[pl.ds(g * 16, 16)]
            av = plsc.load_gather(asrc_v, [s16])
            bv = plsc.load_gather(adst_v, [d16])
            al = av + bv + e16
            al = jnp.where(al >= 0.0, al, 0.2 * al)
            ex = jnp.exp(al)
            exb[pl.ds(g * 16, 16)] = ex
            plsc.addupdate_scatter(denom_v, [d16], ex)
            plsc.addupdate_scatter(deg_v, [d16], ones16)
            plsc.addupdate_scatter(sattr_v, [d16], e16)

        def scale_row(r, carry2):
            exs = plsc.load_gather(exb, [lax.broadcast(r, (16,))])
            for q in range(C // 16):
                rowsb[r, pl.ds(q * 16, 16)] = rowsb[r, pl.ds(q * 16, 16)] * exs
            return carry2

        lax.fori_loop(0, CHUNK, scale_row, 0)
        pltpu.sync_copy(rowsb, outu_sh.at[dstb], add=True)
        return carry

    lax.fori_loop(0, NCHUNK, chunk_body, 0)

    plsc.subcore_barrier()
    pltpu.sync_copy(denom_v, denom_hbm.at[pl.ds(wid * NP, NP)])
    pltpu.sync_copy(deg_v, deg_hbm.at[pl.ds(wid * NP, NP)])
    pltpu.sync_copy(sattr_v, sattr_hbm.at[pl.ds(wid * NP, NP)])
    pltpu.sync_copy(outu_sh.at[pl.ds(sid * ROWS_PT, ROWS_PT), :],
                    outu_hbm.at[cid, pl.ds(sid * ROWS_PT, ROWS_PT), :])


def kernel(x, edge_index, edge_attr, W, b, att_src, att_dst, W_edge, att_edge,
           W_out, b_out):
    src = edge_index[0].astype(jnp.int32)
    dst = edge_index[1].astype(jnp.int32)
    ea2 = edge_attr.reshape(E // 128, 128)
    xl, asrc2, adst2, eac2 = _pre_call(
        x, W, att_src[:, None], att_dst[:, None], W_edge, att_edge[None, :], ea2)
    eac = eac2.reshape(E)
    asrc = asrc2.reshape(N)
    adst = adst2.reshape(N)
    outu_p, denom_p, deg_p, sattr_p = _sc_edge(src, dst, eac, asrc, adst, xl)
    denom_p = denom_p.reshape(NW, NP)[:, :N]
    deg_p = deg_p.reshape(NW, NP)[:, :N]
    sattr_p = sattr_p.reshape(NW, NP)[:, :N]
    res = _post_call(outu_p[0, :N], outu_p[1, :N], denom_p, deg_p, sattr_p, xl,
                     asrc2, adst2, b[None, :], W_out, b_out[:, None],
                     jnp.ones((NW, 1), jnp.float32))
    return res[:, :, None]


# speedup vs baseline: 48.8037x; 1.9378x over previous
import functools

import jax
import jax.numpy as jnp
from jax import lax
from jax.experimental import pallas as pl
from jax.experimental.pallas import tpu as pltpu
from jax.experimental.pallas import tpu_sc as plsc

N = 10000
E = 320000
C = 64
L = 4
NC = 2
NS = 16
NW = NC * NS
EPW = E // NW
CHUNK = 80
NCHUNK = EPW // CHUNK
NP = 10240
ROWS_PT = NP // NS
ZROWS = 128


def _pre_body(x_ref, w_ref, asv_ref, adv_ref, we_ref, ae_ref, ea_ref,
              xl_ref, asrc_ref, adst_ref, eac_ref):
    xl = jnp.dot(x_ref[...], w_ref[...], preferred_element_type=jnp.float32)
    xl_ref[...] = xl
    asrc_ref[...] = jnp.dot(xl, asv_ref[...], preferred_element_type=jnp.float32)
    adst_ref[...] = jnp.dot(xl, adv_ref[...], preferred_element_type=jnp.float32)
    c = jnp.sum(we_ref[...] * ae_ref[...])
    eac_ref[...] = ea_ref[...] * c


_pre_call = pl.pallas_call(
    _pre_body,
    out_shape=[
        jax.ShapeDtypeStruct((N, C), jnp.float32),
        jax.ShapeDtypeStruct((N, 1), jnp.float32),
        jax.ShapeDtypeStruct((N, 1), jnp.float32),
        jax.ShapeDtypeStruct((E // 128, 128), jnp.float32),
    ],
)


def _post_body(oa_ref, ob_ref, dp_ref, gp_ref, sp_ref, xl_ref, as_ref, ad_ref,
               b_ref, wo_ref, bo_ref, ones_ref, res_ref):
    cdims = (((0,), (0,)), ((), ()))
    ones = ones_ref[...]
    dn = lax.dot_general(dp_ref[...], ones, cdims, preferred_element_type=jnp.float32)
    dg = lax.dot_general(gp_ref[...], ones, cdims, preferred_element_type=jnp.float32)
    sa = lax.dot_general(sp_ref[...], ones, cdims, preferred_element_type=jnp.float32)
    loop_c = sa / jnp.maximum(dg, 1.0)
    a_self = as_ref[...] + ad_ref[...] + loop_c
    a_self = jnp.where(a_self >= 0.0, a_self, 0.2 * a_self)
    ex_self = jnp.exp(a_self)
    xl = xl_ref[...]
    outu = oa_ref[...] + ob_ref[...] + ex_self * xl
    out = outu / jnp.maximum(dn + ex_self, 1e-16) + b_ref[...]
    res = lax.dot_general(wo_ref[...], out, (((1,), (1,)), ((), ())),
                          preferred_element_type=jnp.float32)
    res_ref[...] = res + bo_ref[...]


_post_call = pl.pallas_call(
    _post_body,
    out_shape=jax.ShapeDtypeStruct((L, N), jnp.float32),
)


_sc_mesh = plsc.VectorSubcoreMesh(core_axis_name="c", subcore_axis_name="s")


@functools.partial(
    pl.kernel,
    out_type=[
        jax.ShapeDtypeStruct((NC, NP, C), jnp.float32),
        jax.ShapeDtypeStruct((NW * NP,), jnp.float32),
        jax.ShapeDtypeStruct((NW * NP,), jnp.float32),
        jax.ShapeDtypeStruct((NW * NP,), jnp.float32),
    ],
    mesh=_sc_mesh,
    compiler_params=pltpu.CompilerParams(
        needs_layout_passes=False, use_tc_tiling_on_sc=False),
    scratch_types=[
        pltpu.VMEM((N,), jnp.float32),
        pltpu.VMEM((N,), jnp.float32),
        pltpu.VMEM((N,), jnp.float32),
        pltpu.VMEM((N,), jnp.float32),
        pltpu.VMEM((N,), jnp.float32),
        pltpu.VMEM((EPW,), jnp.int32),
        pltpu.VMEM((EPW,), jnp.int32),
        [pltpu.VMEM((CHUNK,), jnp.int32)] * 2,
        [pltpu.VMEM((CHUNK,), jnp.int32)] * 2,
        [pltpu.VMEM((CHUNK,), jnp.float32)] * 2,
        [pltpu.VMEM((CHUNK, C), jnp.float32)] * 2,
        pltpu.VMEM((CHUNK,), jnp.float32),
        pltpu.VMEM_SHARED((NP, C), jnp.float32),
        [pltpu.SemaphoreType.DMA] * 2,
        [pltpu.SemaphoreType.DMA] * 2,
    ],
)
def _sc_edge(src_hbm, dst_hbm, eac_hbm, asrc_hbm, adst_hbm, xl_hbm,
             outu_hbm, denom_hbm, deg_hbm, sattr_hbm,
             asrc_v, adst_v, denom_v, deg_v, sattr_v,
             src_all, dst_all, srcb, dstb, eacb, rowsb, exb,
             outu_sh, semg, seme):
    cid = lax.axis_index("c")
    sid = lax.axis_index("s")
    wid = sid * NC + cid
    ebase = wid * EPW

    pltpu.sync_copy(asrc_hbm, asrc_v)
    pltpu.sync_copy(adst_hbm, adst_v)
    pltpu.sync_copy(src_hbm.at[pl.ds(ebase, EPW)], src_all)
    pltpu.sync_copy(dst_hbm.at[pl.ds(ebase, EPW)], dst_all)

    zeros16 = jnp.zeros((16,), jnp.float32)
    ones16 = jnp.full((16,), 1.0, jnp.float32)

    def zero_locals(i, carry):
        denom_v[pl.ds(i * 16, 16)] = zeros16
        deg_v[pl.ds(i * 16, 16)] = zeros16
        sattr_v[pl.ds(i * 16, 16)] = zeros16
        return carry

    lax.fori_loop(0, N // 16, zero_locals, 0)

    def zero_rows(r, carry):
        for q in range(C // 16):
            rowsb[0][r, pl.ds(q * 16, 16)] = zeros16
        return carry

    lax.fori_loop(0, CHUNK, zero_rows, 0)

    def zero_shared(j, carry):
        pltpu.sync_copy(rowsb[0],
                        outu_sh.at[pl.ds(sid * ROWS_PT + j * CHUNK, CHUNK), :])
        return carry

    lax.fori_loop(0, ROWS_PT // CHUNK, zero_shared, 0)
    plsc.subcore_barrier()

    def start_gather(k, b):
        base = k * CHUNK
        pltpu.async_copy(eac_hbm.at[pl.ds(ebase + base, CHUNK)], eacb[b], seme[b])
        for g in range(CHUNK // 16):
            srcb[b][pl.ds(g * 16, 16)] = src_all[pl.ds(base + g * 16, 16)]
            dstb[b][pl.ds(g * 16, 16)] = dst_all[pl.ds(base + g * 16, 16)]
        pltpu.async_copy(xl_hbm.at[srcb[b]], rowsb[b], semg[b])

    def wait_gather(k, b):
        pltpu.make_async_copy(
            eac_hbm.at[pl.ds(ebase + k * CHUNK, CHUNK)], eacb[b], seme[b]).wait()
        pltpu.make_async_copy(xl_hbm.at[srcb[b]], rowsb[b], semg[b]).wait()

    def compute_chunk(k, b):
        base = k * CHUNK
        for g in range(CHUNK // 16):
            s16 = srcb[b][pl.ds(g * 16, 16)]
            d16 = dstb[b][pl.ds(g * 16, 16)]
            e16 = eacb[b][pl.ds(g * 16, 16)]
            av = plsc.load_gather(asrc_v, [s16])
            bv = plsc.load_gather(adst_v, [d16])
            al = av + bv + e16
            al = jnp.where(al >= 0.0, al, 0.2 * al)
            ex = jnp.exp(al)
            exb[pl.ds(g * 16, 16)] = ex
            plsc.addupdate_scatter(denom_v, [d16], ex)
            plsc.addupdate_scatter(deg_v, [d16], ones16)
            plsc.addupdate_scatter(sattr_v, [d16], e16)

        def scale_row(r, carry2):
            exs = plsc.load_gather(exb, [lax.broadcast(r, (16,))])
            for q in range(C // 16):
                rowsb[b][r, pl.ds(q * 16, 16)] = rowsb[b][r, pl.ds(q * 16, 16)] * exs
            return carry2

        lax.fori_loop(0, CHUNK, scale_row, 0)
        pltpu.sync_copy(rowsb[b], outu_sh.at[dstb[b]], add=True)

    start_gather(0, 0)

    def pair_body(j, carry):
        k0 = 2 * j
        start_gather(k0 + 1, 1)
        wait_gather(k0, 0)
        compute_chunk(k0, 0)
        start_gather(k0 + 2, 0)
        wait_gather(k0 + 1, 1)
        compute_chunk(k0 + 1, 1)
        return carry

    lax.fori_loop(0, (NCHUNK - 1) // 2, pair_body, 0)
    wait_gather(NCHUNK - 1, 0)
    compute_chunk(NCHUNK - 1, 0)

    plsc.subcore_barrier()
    pltpu.sync_copy(denom_v, denom_hbm.at[pl.ds(wid * NP, N)])
    pltpu.sync_copy(deg_v, deg_hbm.at[pl.ds(wid * NP, N)])
    pltpu.sync_copy(sattr_v, sattr_hbm.at[pl.ds(wid * NP, N)])
    pltpu.sync_copy(outu_sh.at[pl.ds(sid * ROWS_PT, ROWS_PT), :],
                    outu_hbm.at[cid, pl.ds(sid * ROWS_PT, ROWS_PT), :])


def kernel(x, edge_index, edge_attr, W, b, att_src, att_dst, W_edge, att_edge,
           W_out, b_out):
    src = edge_index[0].astype(jnp.int32)
    dst = edge_index[1].astype(jnp.int32)
    ea2 = edge_attr.reshape(E // 128, 128)
    xl, asrc2, adst2, eac2 = _pre_call(
        x, W, att_src[:, None], att_dst[:, None], W_edge, att_edge[None, :], ea2)
    eac = eac2.reshape(E)
    asrc = asrc2.reshape(N)
    adst = adst2.reshape(N)
    outu_p, denom_p, deg_p, sattr_p = _sc_edge(src, dst, eac, asrc, adst, xl)
    denom_p = denom_p.reshape(NW, NP)[:, :N]
    deg_p = deg_p.reshape(NW, NP)[:, :N]
    sattr_p = sattr_p.reshape(NW, NP)[:, :N]
    res = _post_call(outu_p[0, :N], outu_p[1, :N], denom_p, deg_p, sattr_p, xl,
                     asrc2, adst2, b[None, :], W_out, b_out[:, None],
                     jnp.ones((NW, 1), jnp.float32))
    return res[:, :, None]


# speedup vs baseline: 48.9455x; 1.0029x over previous
import functools

import jax
import jax.numpy as jnp
from jax import lax
from jax.experimental import pallas as pl
from jax.experimental.pallas import tpu as pltpu
from jax.experimental.pallas import tpu_sc as plsc

N = 10000
E = 320000
C = 64
L = 4
NC = 2
NS = 16
NW = NC * NS
EPW = E // NW
CHUNK = 80
NCHUNK = EPW // CHUNK
NP = 10240
ROWS_PT = NP // NS
ZROWS = 128


def _pre_body(x_ref, w_ref, asv_ref, adv_ref, we_ref, ae_ref, ea_ref,
              xl_ref, asrc_ref, adst_ref, eac_ref):
    xl = jnp.dot(x_ref[...], w_ref[...], preferred_element_type=jnp.float32)
    xl_ref[...] = xl
    asrc_ref[...] = jnp.dot(xl, asv_ref[...], preferred_element_type=jnp.float32)
    adst_ref[...] = jnp.dot(xl, adv_ref[...], preferred_element_type=jnp.float32)
    c = jnp.sum(we_ref[...] * ae_ref[...])
    eac_ref[...] = ea_ref[...] * c


_pre_call = pl.pallas_call(
    _pre_body,
    out_shape=[
        jax.ShapeDtypeStruct((N, C), jnp.float32),
        jax.ShapeDtypeStruct((N, 1), jnp.float32),
        jax.ShapeDtypeStruct((N, 1), jnp.float32),
        jax.ShapeDtypeStruct((E // 128, 128), jnp.float32),
    ],
)


def _post_body(oa_ref, ob_ref, dp_ref, gp_ref, sp_ref, xl_ref, as_ref, ad_ref,
               b_ref, wo_ref, bo_ref, ones_ref, res_ref):
    cdims = (((0,), (0,)), ((), ()))
    ones = ones_ref[...]
    dn = lax.dot_general(dp_ref[...], ones, cdims, preferred_element_type=jnp.float32)
    dg = lax.dot_general(gp_ref[...], ones, cdims, preferred_element_type=jnp.float32)
    sa = lax.dot_general(sp_ref[...], ones, cdims, preferred_element_type=jnp.float32)
    loop_c = sa / jnp.maximum(dg, 1.0)
    a_self = as_ref[...] + ad_ref[...] + loop_c
    a_self = jnp.where(a_self >= 0.0, a_self, 0.2 * a_self)
    ex_self = jnp.exp(a_self)
    xl = xl_ref[...]
    outu = oa_ref[...] + ob_ref[...] + ex_self * xl
    out = outu / jnp.maximum(dn + ex_self, 1e-16) + b_ref[...]
    res = lax.dot_general(wo_ref[...], out, (((1,), (1,)), ((), ())),
                          preferred_element_type=jnp.float32)
    res_ref[...] = res + bo_ref[...]


_post_call = pl.pallas_call(
    _post_body,
    out_shape=jax.ShapeDtypeStruct((L, N), jnp.float32),
)


_sc_mesh = plsc.VectorSubcoreMesh(core_axis_name="c", subcore_axis_name="s")


@functools.partial(
    pl.kernel,
    out_type=[
        jax.ShapeDtypeStruct((NC, NP, C), jnp.float32),
        jax.ShapeDtypeStruct((NW * NP,), jnp.float32),
        jax.ShapeDtypeStruct((NW * NP,), jnp.float32),
        jax.ShapeDtypeStruct((NW * NP,), jnp.float32),
    ],
    mesh=_sc_mesh,
    compiler_params=pltpu.CompilerParams(
        needs_layout_passes=False, use_tc_tiling_on_sc=False),
    scratch_types=[
        pltpu.VMEM((N,), jnp.float32),
        pltpu.VMEM((N,), jnp.float32),
        pltpu.VMEM((N,), jnp.float32),
        pltpu.VMEM((N,), jnp.float32),
        pltpu.VMEM((N,), jnp.float32),
        pltpu.VMEM((EPW,), jnp.int32),
        pltpu.VMEM((EPW,), jnp.int32),
        [pltpu.VMEM((CHUNK,), jnp.int32)] * 2,
        [pltpu.VMEM((CHUNK,), jnp.int32)] * 2,
        [pltpu.VMEM((CHUNK,), jnp.float32)] * 2,
        [pltpu.VMEM((CHUNK, C), jnp.float32)] * 2,
        pltpu.VMEM((CHUNK,), jnp.float32),
        pltpu.VMEM_SHARED((NP, C), jnp.float32),
        [pltpu.SemaphoreType.DMA] * 2,
        [pltpu.SemaphoreType.DMA] * 2,
        [pltpu.SemaphoreType.DMA] * 2,
    ],
)
def _sc_edge(src_hbm, dst_hbm, eac_hbm, asrc_hbm, adst_hbm, xl_hbm,
             outu_hbm, denom_hbm, deg_hbm, sattr_hbm,
             asrc_v, adst_v, denom_v, deg_v, sattr_v,
             src_all, dst_all, srcb, dstb, eacb, rowsb, exb,
             outu_sh, semg, seme, sems):
    cid = lax.axis_index("c")
    sid = lax.axis_index("s")
    wid = sid * NC + cid
    ebase = wid * EPW

    pltpu.sync_copy(asrc_hbm, asrc_v)
    pltpu.sync_copy(adst_hbm, adst_v)
    pltpu.sync_copy(src_hbm.at[pl.ds(ebase, EPW)], src_all)
    pltpu.sync_copy(dst_hbm.at[pl.ds(ebase, EPW)], dst_all)

    zeros16 = jnp.zeros((16,), jnp.float32)
    ones16 = jnp.full((16,), 1.0, jnp.float32)

    def zero_locals(i, carry):
        denom_v[pl.ds(i * 16, 16)] = zeros16
        deg_v[pl.ds(i * 16, 16)] = zeros16
        sattr_v[pl.ds(i * 16, 16)] = zeros16
        return carry

    lax.fori_loop(0, N // 16, zero_locals, 0)

    def zero_rows(r, carry):
        for q in range(C // 16):
            rowsb[0][r, pl.ds(q * 16, 16)] = zeros16
        return carry

    lax.fori_loop(0, CHUNK, zero_rows, 0)

    def zero_shared(j, carry):
        pltpu.sync_copy(rowsb[0],
                        outu_sh.at[pl.ds(sid * ROWS_PT + j * CHUNK, CHUNK), :])
        return carry

    lax.fori_loop(0, ROWS_PT // CHUNK, zero_shared, 0)
    plsc.subcore_barrier()

    def start_gather(k, b):
        base = k * CHUNK
        pltpu.async_copy(eac_hbm.at[pl.ds(ebase + base, CHUNK)], eacb[b], seme[b])
        for g in range(CHUNK // 16):
            srcb[b][pl.ds(g * 16, 16)] = src_all[pl.ds(base + g * 16, 16)]
            dstb[b][pl.ds(g * 16, 16)] = dst_all[pl.ds(base + g * 16, 16)]
        pltpu.async_copy(xl_hbm.at[srcb[b]], rowsb[b], semg[b])

    def wait_eac(k, b):
        pltpu.make_async_copy(
            eac_hbm.at[pl.ds(ebase + k * CHUNK, CHUNK)], eacb[b], seme[b]).wait()

    def wait_rows(b):
        pltpu.make_async_copy(xl_hbm.at[srcb[b]], rowsb[b], semg[b]).wait()

    def alpha_part(b):
        for g in range(CHUNK // 16):
            s16 = srcb[b][pl.ds(g * 16, 16)]
            d16 = dstb[b][pl.ds(g * 16, 16)]
            e16 = eacb[b][pl.ds(g * 16, 16)]
            av = plsc.load_gather(asrc_v, [s16])
            bv = plsc.load_gather(adst_v, [d16])
            al = av + bv + e16
            al = jnp.where(al >= 0.0, al, 0.2 * al)
            ex = jnp.exp(al)
            exb[pl.ds(g * 16, 16)] = ex
            plsc.addupdate_scatter(denom_v, [d16], ex)
            plsc.addupdate_scatter(deg_v, [d16], ones16)
            plsc.addupdate_scatter(sattr_v, [d16], e16)

    def scale_part(b):
        def scale_row(r, carry2):
            exs = plsc.load_gather(exb, [lax.broadcast(r, (16,))])
            for q in range(C // 16):
                rowsb[b][r, pl.ds(q * 16, 16)] = rowsb[b][r, pl.ds(q * 16, 16)] * exs
            return carry2

        lax.fori_loop(0, CHUNK, scale_row, 0)

    def start_scatter(b):
        pltpu.async_copy(rowsb[b], outu_sh.at[dstb[b]], sems[b], add=True)

    def wait_scatter(b):
        pltpu.make_async_copy(rowsb[b], outu_sh.at[dstb[b]], sems[b]).wait()

    start_gather(0, 0)
    start_gather(1, 1)
    wait_eac(0, 0)
    alpha_part(0)
    wait_rows(0)
    scale_part(0)
    start_scatter(0)

    def body(k, b):
        nb = 1 - b
        wait_scatter(nb)
        start_gather(k + 1, nb)
        wait_eac(k, b)
        alpha_part(b)
        wait_rows(b)
        scale_part(b)
        start_scatter(b)

    def pair_body(j, carry):
        body(2 * j + 1, 1)
        body(2 * j + 2, 0)
        return carry

    lax.fori_loop(0, (NCHUNK - 3) // 2, pair_body, 0)
    body(NCHUNK - 2, 1)
    wait_eac(NCHUNK - 1, 0)
    alpha_part(0)
    wait_rows(0)
    scale_part(0)
    pltpu.sync_copy(rowsb[0], outu_sh.at[dstb[0]], add=True)
    wait_scatter(1)

    plsc.subcore_barrier()
    pltpu.sync_copy(denom_v, denom_hbm.at[pl.ds(wid * NP, N)])
    pltpu.sync_copy(deg_v, deg_hbm.at[pl.ds(wid * NP, N)])
    pltpu.sync_copy(sattr_v, sattr_hbm.at[pl.ds(wid * NP, N)])
    pltpu.sync_copy(outu_sh.at[pl.ds(sid * ROWS_PT, ROWS_PT), :],
                    outu_hbm.at[cid, pl.ds(sid * ROWS_PT, ROWS_PT), :])


def kernel(x, edge_index, edge_attr, W, b, att_src, att_dst, W_edge, att_edge,
           W_out, b_out):
    src = edge_index[0].astype(jnp.int32)
    dst = edge_index[1].astype(jnp.int32)
    ea2 = edge_attr.reshape(E // 128, 128)
    xl, asrc2, adst2, eac2 = _pre_call(
        x, W, att_src[:, None], att_dst[:, None], W_edge, att_edge[None, :], ea2)
    eac = eac2.reshape(E)
    asrc = asrc2.reshape(N)
    adst = adst2.reshape(N)
    outu_p, denom_p, deg_p, sattr_p = _sc_edge(src, dst, eac, asrc, adst, xl)
    denom_p = denom_p.reshape(NW, NP)[:, :N]
    deg_p = deg_p.reshape(NW, NP)[:, :N]
    sattr_p = sattr_p.reshape(NW, NP)[:, :N]
    res = _post_call(outu_p[0, :N], outu_p[1, :N], denom_p, deg_p, sattr_p, xl,
                     asrc2, adst2, b[None, :], W_out, b_out[:, None],
                     jnp.ones((NW, 1), jnp.float32))
    return res[:, :, None]
